# baseline (speedup 1.0000x reference)
import functools

import jax
import jax.numpy as jnp
from jax import lax
from jax.experimental import pallas as pl

_NUM_CODES = 8192
_D = 32
_COMMIT = 0.1

_TB = 512
_CT = 1024
_CHUNK = 4096


def _vq_body(flat_ref, emb_ref, xsq_ref, esq_ref, codes_ref, zq_ref, loss_ref):
    x = flat_ref[...]
    xsq = xsq_ref[0, 0, :][:, None]

    run_v = jnp.full((_TB,), jnp.inf, jnp.float32)
    run_exact = jnp.full((_TB,), jnp.inf, jnp.float32)
    run_i = jnp.zeros((_TB,), jnp.int32)
    gathered = jnp.zeros((_TB, _D), jnp.float32)

    for t in range(_NUM_CODES // _CT):
        e = emb_ref[pl.ds(t * _CT, _CT), :]
        esq = esq_ref[0, pl.ds(t * _CT, _CT)]
        m = lax.dot_general(x, e, (((1,), (1,)), ((), ())),
                            preferred_element_type=jnp.float32,
                            precision=lax.Precision.DEFAULT)
        scores = (xsq - 2.0 * m) + esq[None, :]
        iota = lax.broadcasted_iota(jnp.int32, (_TB, _CT), 1)
        tmin = jnp.min(scores, axis=1, keepdims=True)
        targ = jnp.min(jnp.where(scores == tmin, iota, _NUM_CODES), axis=1)
        onehot = (iota == targ[:, None]).astype(jnp.float32)
        g = lax.dot_general(onehot, e, (((1,), (0,)), ((), ())),
                            preferred_element_type=jnp.float32,
                            precision=lax.Precision.HIGHEST)
        better = tmin[:, 0] < run_v
        run_v = jnp.where(better, tmin[:, 0], run_v)
        run_exact = jnp.where(better, tmin[:, 0], run_exact)
        run_i = jnp.where(better, targ + t * _CT, run_i)
        gathered = jnp.where(better[:, None], g, gathered)
        if (t + 1) * _CT % _CHUNK == 0:
            run_v = run_v.astype(jnp.bfloat16).astype(jnp.float32)

    codes_ref[0, 0, :] = run_i
    zq_ref[...] = gathered

    @pl.when(pl.program_id(0) == 0)
    def _():
        loss_ref[...] = jnp.zeros((1, 1), jnp.float32)
    loss_ref[...] += jnp.sum(run_exact).reshape(1, 1)


@functools.partial(jax.jit, static_argnames=())
def kernel(z_e, emb):
    B, L, D = z_e.shape
    n = B * L
    flat = z_e.reshape(n, D)
    nblocks = n // _TB

    xsq = jnp.sum(flat ** 2, axis=1, keepdims=True)
    esq = jnp.sum(emb ** 2, axis=1, keepdims=True).T
    xsq3 = xsq.reshape(nblocks, 1, _TB)

    codes3, zq, loss = pl.pallas_call(
        _vq_body,
        grid=(nblocks,),
        in_specs=[
            pl.BlockSpec((_TB, D), lambda i: (i, 0)),
            pl.BlockSpec((_NUM_CODES, D), lambda i: (0, 0)),
            pl.BlockSpec((1, 1, _TB), lambda i: (i, 0, 0)),
            pl.BlockSpec((1, _NUM_CODES), lambda i: (0, 0)),
        ],
        out_specs=[
            pl.BlockSpec((1, 1, _TB), lambda i: (i, 0, 0)),
            pl.BlockSpec((_TB, D), lambda i: (i, 0)),
            pl.BlockSpec((1, 1), lambda i: (0, 0)),
        ],
        out_shape=[
            jax.ShapeDtypeStruct((nblocks, 1, _TB), jnp.int32),
            jax.ShapeDtypeStruct((n, D), jnp.float32),
            jax.ShapeDtypeStruct((1, 1), jnp.float32),
        ],
    )(flat, emb, xsq3, esq)

    codes = codes3.reshape(B, L)
    z_q = zq.reshape(B, L, D)
    z_q_st = z_e + lax.stop_gradient(z_q - z_e)
    loss_vq = (_COMMIT / (n * D)) * loss[0, 0]
    return (z_q_st, loss_vq, codes)


# speedup vs baseline: 2.7715x; 2.7715x over previous
import functools

import jax
import jax.numpy as jnp
from jax import lax
from jax.experimental import pallas as pl
from jax.experimental.pallas import tpu as pltpu
from jax.experimental.pallas import tpu_sc as plsc

_NUM_CODES = 8192
_D = 32
_COMMIT = 0.1

_TB = 512
_CT = 1024
_CHUNK = 4096


def _vq_body(flat_ref, emb_ref, xsq_ref, esq_ref, codes_ref, loss_ref):
    x = flat_ref[...]
    xsq = xsq_ref[0, 0, :][:, None]

    run_v = jnp.full((_TB,), jnp.inf, jnp.float32)
    run_exact = jnp.full((_TB,), jnp.inf, jnp.float32)
    run_i = jnp.zeros((_TB,), jnp.int32)

    for t in range(_NUM_CODES // _CT):
        e = emb_ref[pl.ds(t * _CT, _CT), :]
        esq = esq_ref[0, pl.ds(t * _CT, _CT)]
        m = lax.dot_general(x, e, (((1,), (1,)), ((), ())),
                            preferred_element_type=jnp.float32,
                            precision=lax.Precision.DEFAULT)
        scores = (xsq - 2.0 * m) + esq[None, :]
        iota = lax.broadcasted_iota(jnp.int32, (_TB, _CT), 1)
        tmin = jnp.min(scores, axis=1, keepdims=True)
        targ = jnp.min(jnp.where(scores == tmin, iota, _NUM_CODES), axis=1)
        better = tmin[:, 0] < run_v
        run_v = jnp.where(better, tmin[:, 0], run_v)
        run_exact = jnp.where(better, tmin[:, 0], run_exact)
        run_i = jnp.where(better, targ + t * _CT, run_i)
        if (t + 1) * _CT % _CHUNK == 0:
            run_v = run_v.astype(jnp.bfloat16).astype(jnp.float32)

    codes_ref[0, 0, :] = run_i

    @pl.when(pl.program_id(0) == 0)
    def _():
        loss_ref[...] = jnp.zeros((1, 1), jnp.float32)
    loss_ref[...] += jnp.sum(run_exact).reshape(1, 1)


def _make_sc_gather(n):
    info = plsc.get_sparse_core_info()
    nw = info.num_cores * info.num_subcores
    b_per_w = n // nw
    nch = b_per_w // 128
    mesh = plsc.VectorSubcoreMesh(core_axis_name="c", subcore_axis_name="s")

    @functools.partial(
        pl.kernel, mesh=mesh,
        out_type=jax.ShapeDtypeStruct((n, 128), jnp.float32),
        scratch_types=[
            pltpu.VMEM((nch, 128), jnp.int32),
            pltpu.VMEM((2, 128, 128), jnp.float32),
            pltpu.SemaphoreType.DMA,
            pltpu.SemaphoreType.DMA,
        ],
    )
    def sc_gather(table_hbm, idx_hbm, out_hbm, idx_v, rows_v, sem0, sem1):
        wid = lax.axis_index("s") * info.num_cores + lax.axis_index("c")
        base = wid * b_per_w
        pltpu.sync_copy(idx_hbm.at[wid], idx_v)
        sems = (sem0, sem1)
        prev = None
        for j in range(nch):
            cur = pltpu.async_copy(table_hbm.at[idx_v.at[j]],
                                   rows_v.at[j % 2], sems[j % 2])
            if prev is not None:
                pj, pc = prev
                pc.wait()
                pltpu.sync_copy(rows_v.at[pj % 2],
                                out_hbm.at[pl.ds(base + pj * 128, 128)])
            prev = (j, cur)
        pj, pc = prev
        pc.wait()
        pltpu.sync_copy(rows_v.at[pj % 2],
                        out_hbm.at[pl.ds(base + pj * 128, 128)])

    return sc_gather, nw, b_per_w


@functools.partial(jax.jit, static_argnames=())
def kernel(z_e, emb):
    B, L, D = z_e.shape
    n = B * L
    flat = z_e.reshape(n, D)
    nblocks = n // _TB

    xsq = jnp.sum(flat ** 2, axis=1, keepdims=True)
    esq = jnp.sum(emb ** 2, axis=1, keepdims=True).T
    xsq3 = xsq.reshape(nblocks, 1, _TB)

    codes3, loss = pl.pallas_call(
        _vq_body,
        grid=(nblocks,),
        in_specs=[
            pl.BlockSpec((_TB, D), lambda i: (i, 0)),
            pl.BlockSpec((_NUM_CODES, D), lambda i: (0, 0)),
            pl.BlockSpec((1, 1, _TB), lambda i: (i, 0, 0)),
            pl.BlockSpec((1, _NUM_CODES), lambda i: (0, 0)),
        ],
        out_specs=[
            pl.BlockSpec((1, 1, _TB), lambda i: (i, 0, 0)),
            pl.BlockSpec((1, 1), lambda i: (0, 0)),
        ],
        out_shape=[
            jax.ShapeDtypeStruct((nblocks, 1, _TB), jnp.int32),
            jax.ShapeDtypeStruct((1, 1), jnp.float32),
        ],
    )(flat, emb, xsq3, esq)

    sc_gather, nw, b_per_w = _make_sc_gather(n)
    idx3 = codes3.reshape(nw, b_per_w // 128, 128)
    emb_pad = jnp.pad(emb, ((0, 0), (0, 128 - _D)))
    zq = sc_gather(emb_pad, idx3)

    codes = codes3.reshape(B, L)
    z_q = zq[:, :_D].reshape(B, L, D)
    z_q_st = z_e + lax.stop_gradient(z_q - z_e)
    loss_vq = (_COMMIT / (n * D)) * loss[0, 0]
    return (z_q_st, loss_vq, codes)


# speedup vs baseline: 2.9212x; 1.0540x over previous
import functools

import jax
import jax.numpy as jnp
from jax import lax
from jax.experimental import pallas as pl
from jax.experimental.pallas import tpu as pltpu
from jax.experimental.pallas import tpu_sc as plsc

_NUM_CODES = 8192
_D = 32
_COMMIT = 0.1

_TB = 512
_CT = 1024
_CHUNK = 4096


def _vq_body(flat_ref, emb_ref, xsq_ref, esq_ref, codes_ref, loss_ref):
    x2 = flat_ref[...] * -2.0
    xsq = xsq_ref[0, 0, :][:, None]

    run_v = jnp.full((_TB,), jnp.inf, jnp.float32)
    run_exact = jnp.full((_TB,), jnp.inf, jnp.float32)
    run_i = jnp.zeros((_TB,), jnp.int32)

    for t in range(_NUM_CODES // _CT):
        e = emb_ref[pl.ds(t * _CT, _CT), :]
        esq = esq_ref[0, pl.ds(t * _CT, _CT)]
        m2 = lax.dot_general(x2, e, (((1,), (1,)), ((), ())),
                             preferred_element_type=jnp.float32,
                             precision=lax.Precision.DEFAULT)
        scores = (xsq + m2) + esq[None, :]
        iota = lax.broadcasted_iota(jnp.int32, (_TB, _CT), 1)
        tmin = jnp.min(scores, axis=1, keepdims=True)
        targ = jnp.min(jnp.where(scores == tmin, iota, _NUM_CODES), axis=1)
        better = tmin[:, 0] < run_v
        run_v = jnp.where(better, tmin[:, 0], run_v)
        run_exact = jnp.where(better, tmin[:, 0], run_exact)
        run_i = jnp.where(better, targ + t * _CT, run_i)
        if (t + 1) * _CT % _CHUNK == 0:
            run_v = run_v.astype(jnp.bfloat16).astype(jnp.float32)

    codes_ref[0, 0, :] = run_i

    @pl.when(pl.program_id(0) == 0)
    def _():
        loss_ref[...] = jnp.zeros((1, 1), jnp.float32)
    loss_ref[...] += jnp.sum(run_exact).reshape(1, 1)


def _make_sc_gather(n):
    info = plsc.get_sparse_core_info()
    nw = info.num_cores * info.num_subcores
    b_per_w = n // nw
    nch = b_per_w // 128
    mesh = plsc.VectorSubcoreMesh(core_axis_name="c", subcore_axis_name="s")

    @functools.partial(
        pl.kernel, mesh=mesh,
        out_type=jax.ShapeDtypeStruct((n, 128), jnp.float32),
        scratch_types=[
            pltpu.VMEM((nch, 128), jnp.int32),
            pltpu.VMEM((2, 128, 128), jnp.float32),
            pltpu.SemaphoreType.DMA,
            pltpu.SemaphoreType.DMA,
        ],
    )
    def sc_gather(table_hbm, idx_hbm, out_hbm, idx_v, rows_v, sem0, sem1):
        wid = lax.axis_index("s") * info.num_cores + lax.axis_index("c")
        base = wid * b_per_w
        pltpu.sync_copy(idx_hbm.at[wid], idx_v)
        sems = (sem0, sem1)
        prev = None
        for j in range(nch):
            cur = pltpu.async_copy(table_hbm.at[idx_v.at[j]],
                                   rows_v.at[j % 2], sems[j % 2])
            if prev is not None:
                pj, pc = prev
                pc.wait()
                pltpu.sync_copy(rows_v.at[pj % 2],
                                out_hbm.at[pl.ds(base + pj * 128, 128)])
            prev = (j, cur)
        pj, pc = prev
        pc.wait()
        pltpu.sync_copy(rows_v.at[pj % 2],
                        out_hbm.at[pl.ds(base + pj * 128, 128)])

    return sc_gather, nw, b_per_w


@functools.partial(jax.jit, static_argnames=())
def kernel(z_e, emb):
    B, L, D = z_e.shape
    n = B * L
    flat = z_e.reshape(n, D)
    nblocks = n // _TB

    xsq = jnp.sum(flat ** 2, axis=1, keepdims=True)
    esq = jnp.sum(emb ** 2, axis=1, keepdims=True).T
    xsq3 = xsq.reshape(nblocks, 1, _TB)

    codes3, loss = pl.pallas_call(
        _vq_body,
        grid=(nblocks,),
        in_specs=[
            pl.BlockSpec((_TB, D), lambda i: (i, 0)),
            pl.BlockSpec((_NUM_CODES, D), lambda i: (0, 0)),
            pl.BlockSpec((1, 1, _TB), lambda i: (i, 0, 0)),
            pl.BlockSpec((1, _NUM_CODES), lambda i: (0, 0)),
        ],
        out_specs=[
            pl.BlockSpec((1, 1, _TB), lambda i: (i, 0, 0)),
            pl.BlockSpec((1, 1), lambda i: (0, 0)),
        ],
        out_shape=[
            jax.ShapeDtypeStruct((nblocks, 1, _TB), jnp.int32),
            jax.ShapeDtypeStruct((1, 1), jnp.float32),
        ],
    )(flat, emb, xsq3, esq)

    sc_gather, nw, b_per_w = _make_sc_gather(n)
    idx3 = codes3.reshape(nw, b_per_w // 128, 128)
    emb_pad = jnp.pad(emb, ((0, 0), (0, 128 - _D)))
    zq = sc_gather(emb_pad, idx3)

    codes = codes3.reshape(B, L)
    z_q = zq[:, :_D].reshape(B, L, D)
    z_q_st = z_e + lax.stop_gradient(z_q - z_e)
    loss_vq = (_COMMIT / (n * D)) * loss[0, 0]
    return (z_q_st, loss_vq, codes)


# speedup vs baseline: 3.8816x; 1.3288x over previous
import functools

import jax
import jax.numpy as jnp
from jax import lax
from jax.experimental import pallas as pl
from jax.experimental.pallas import tpu as pltpu
from jax.experimental.pallas import tpu_sc as plsc

_NUM_CODES = 8192
_D = 32
_COMMIT = 0.1

_TB = 512
_CT = 1024
_CHUNK = 4096


def _vq_body(flat_ref, emb_ref, xsq_ref, esq_ref, codes_ref, loss_ref):
    x2 = flat_ref[...] * -2.0
    xsq = xsq_ref[0, 0, :][:, None]

    run_v = jnp.full((_TB,), jnp.inf, jnp.float32)
    run_exact = jnp.full((_TB,), jnp.inf, jnp.float32)
    run_i = jnp.zeros((_TB,), jnp.int32)
    lane = lax.broadcasted_iota(jnp.int32, (_TB, 128), 1).astype(jnp.float32)

    for c in range(_NUM_CODES // _CHUNK):
        vbest = jnp.full((_TB, 128), jnp.inf, jnp.float32)
        vblk = jnp.zeros((_TB, 128), jnp.float32)
        for t in range(_CHUNK // _CT):
            off = c * _CHUNK + t * _CT
            e = emb_ref[pl.ds(off, _CT), :]
            esq = esq_ref[0, pl.ds(off, _CT)]
            m2 = lax.dot_general(x2, e, (((1,), (1,)), ((), ())),
                                 preferred_element_type=jnp.float32,
                                 precision=lax.Precision.DEFAULT)
            scores = (xsq + m2) + esq[None, :]
            for k in range(_CT // 128):
                s = scores[:, k * 128:(k + 1) * 128]
                better = s < vbest
                vbest = jnp.where(better, s, vbest)
                vblk = jnp.where(better, jnp.float32(t * 8 + k), vblk)
        vm = jnp.min(vbest, axis=1, keepdims=True)
        cand = jnp.where(vbest == vm, vblk * 128.0 + lane,
                         jnp.float32(_NUM_CODES))
        g = jnp.min(cand, axis=1)
        cmin = vm[:, 0]
        better = cmin < run_v
        run_exact = jnp.where(better, cmin, run_exact)
        run_i = jnp.where(better, g.astype(jnp.int32) + c * _CHUNK, run_i)
        run_v = jnp.where(better, cmin, run_v)
        run_v = run_v.astype(jnp.bfloat16).astype(jnp.float32)

    codes_ref[0, 0, :] = run_i

    @pl.when(pl.program_id(0) == 0)
    def _():
        loss_ref[...] = jnp.zeros((1, 1), jnp.float32)
    loss_ref[...] += jnp.sum(run_exact).reshape(1, 1)


def _make_sc_gather(n):
    info = plsc.get_sparse_core_info()
    nw = info.num_cores * info.num_subcores
    b_per_w = n // nw
    nch = b_per_w // 128
    mesh = plsc.VectorSubcoreMesh(core_axis_name="c", subcore_axis_name="s")

    @functools.partial(
        pl.kernel, mesh=mesh,
        out_type=jax.ShapeDtypeStruct((n, 128), jnp.float32),
        scratch_types=[
            pltpu.VMEM((nch, 128), jnp.int32),
            pltpu.VMEM((2, 128, 128), jnp.float32),
            pltpu.SemaphoreType.DMA,
            pltpu.SemaphoreType.DMA,
        ],
    )
    def sc_gather(table_hbm, idx_hbm, out_hbm, idx_v, rows_v, sem0, sem1):
        wid = lax.axis_index("s") * info.num_cores + lax.axis_index("c")
        base = wid * b_per_w
        pltpu.sync_copy(idx_hbm.at[wid], idx_v)
        sems = (sem0, sem1)
        prev = None
        for j in range(nch):
            cur = pltpu.async_copy(table_hbm.at[idx_v.at[j]],
                                   rows_v.at[j % 2], sems[j % 2])
            if prev is not None:
                pj, pc = prev
                pc.wait()
                pltpu.sync_copy(rows_v.at[pj % 2],
                                out_hbm.at[pl.ds(base + pj * 128, 128)])
            prev = (j, cur)
        pj, pc = prev
        pc.wait()
        pltpu.sync_copy(rows_v.at[pj % 2],
                        out_hbm.at[pl.ds(base + pj * 128, 128)])

    return sc_gather, nw, b_per_w


@functools.partial(jax.jit, static_argnames=())
def kernel(z_e, emb):
    B, L, D = z_e.shape
    n = B * L
    flat = z_e.reshape(n, D)
    nblocks = n // _TB

    xsq = jnp.sum(flat ** 2, axis=1, keepdims=True)
    esq = jnp.sum(emb ** 2, axis=1, keepdims=True).T
    xsq3 = xsq.reshape(nblocks, 1, _TB)

    codes3, loss = pl.pallas_call(
        _vq_body,
        grid=(nblocks,),
        in_specs=[
            pl.BlockSpec((_TB, D), lambda i: (i, 0)),
            pl.BlockSpec((_NUM_CODES, D), lambda i: (0, 0)),
            pl.BlockSpec((1, 1, _TB), lambda i: (i, 0, 0)),
            pl.BlockSpec((1, _NUM_CODES), lambda i: (0, 0)),
        ],
        out_specs=[
            pl.BlockSpec((1, 1, _TB), lambda i: (i, 0, 0)),
            pl.BlockSpec((1, 1), lambda i: (0, 0)),
        ],
        out_shape=[
            jax.ShapeDtypeStruct((nblocks, 1, _TB), jnp.int32),
            jax.ShapeDtypeStruct((1, 1), jnp.float32),
        ],
    )(flat, emb, xsq3, esq)

    sc_gather, nw, b_per_w = _make_sc_gather(n)
    idx3 = codes3.reshape(nw, b_per_w // 128, 128)
    emb_pad = jnp.pad(emb, ((0, 0), (0, 128 - _D)))
    zq = sc_gather(emb_pad, idx3)

    codes = codes3.reshape(B, L)
    z_q = zq[:, :_D].reshape(B, L, D)
    z_q_st = z_e + lax.stop_gradient(z_q - z_e)
    loss_vq = (_COMMIT / (n * D)) * loss[0, 0]
    return (z_q_st, loss_vq, codes)


# speedup vs baseline: 4.0221x; 1.0362x over previous
import functools

import jax
import jax.numpy as jnp
from jax import lax
from jax.experimental import pallas as pl
from jax.experimental.pallas import tpu as pltpu
from jax.experimental.pallas import tpu_sc as plsc

_NUM_CODES = 8192
_D = 32
_COMMIT = 0.1

_TB = 1024
_CT = 1024
_CHUNK = 4096


def _vq_body(flat_ref, emb_ref, xsq_ref, esq_ref, codes_ref, loss_ref):
    x2 = flat_ref[...] * -2.0
    xsq = xsq_ref[0, 0, :][:, None]

    run_v = jnp.full((_TB,), jnp.inf, jnp.float32)
    run_exact = jnp.full((_TB,), jnp.inf, jnp.float32)
    run_i = jnp.zeros((_TB,), jnp.int32)
    lane = lax.broadcasted_iota(jnp.int32, (_TB, 128), 1).astype(jnp.float32)

    for c in range(_NUM_CODES // _CHUNK):
        vbest = jnp.full((_TB, 128), jnp.inf, jnp.float32)
        vblk = jnp.zeros((_TB, 128), jnp.float32)
        for t in range(_CHUNK // _CT):
            off = c * _CHUNK + t * _CT
            e = emb_ref[pl.ds(off, _CT), :]
            esq = esq_ref[0, pl.ds(off, _CT)]
            m2 = lax.dot_general(x2, e, (((1,), (1,)), ((), ())),
                                 preferred_element_type=jnp.float32,
                                 precision=lax.Precision.DEFAULT)
            scores = (xsq + m2) + esq[None, :]
            for k in range(_CT // 128):
                s = scores[:, k * 128:(k + 1) * 128]
                better = s < vbest
                vbest = jnp.where(better, s, vbest)
                vblk = jnp.where(better, jnp.float32(t * 8 + k), vblk)
        vm = jnp.min(vbest, axis=1, keepdims=True)
        cand = jnp.where(vbest == vm, vblk * 128.0 + lane,
                         jnp.float32(_NUM_CODES))
        g = jnp.min(cand, axis=1)
        cmin = vm[:, 0]
        better = cmin < run_v
        run_exact = jnp.where(better, cmin, run_exact)
        run_i = jnp.where(better, g.astype(jnp.int32) + c * _CHUNK, run_i)
        run_v = jnp.where(better, cmin, run_v)
        run_v = run_v.astype(jnp.bfloat16).astype(jnp.float32)

    codes_ref[0, 0, :] = run_i

    @pl.when(pl.program_id(0) == 0)
    def _():
        loss_ref[...] = jnp.zeros((1, 1), jnp.float32)
    loss_ref[...] += jnp.sum(run_exact).reshape(1, 1)


def _make_sc_gather(n):
    info = plsc.get_sparse_core_info()
    nw = info.num_cores * info.num_subcores
    b_per_w = n // nw
    nch = b_per_w // 128
    mesh = plsc.VectorSubcoreMesh(core_axis_name="c", subcore_axis_name="s")

    @functools.partial(
        pl.kernel, mesh=mesh,
        out_type=jax.ShapeDtypeStruct((n, 128), jnp.float32),
        scratch_types=[
            pltpu.VMEM((nch, 128), jnp.int32),
            pltpu.VMEM((2, 128, 128), jnp.float32),
            pltpu.SemaphoreType.DMA,
            pltpu.SemaphoreType.DMA,
        ],
    )
    def sc_gather(table_hbm, idx_hbm, out_hbm, idx_v, rows_v, sem0, sem1):
        wid = lax.axis_index("s") * info.num_cores + lax.axis_index("c")
        base = wid * b_per_w
        pltpu.sync_copy(idx_hbm.at[wid], idx_v)
        sems = (sem0, sem1)
        prev = None
        for j in range(nch):
            cur = pltpu.async_copy(table_hbm.at[idx_v.at[j]],
                                   rows_v.at[j % 2], sems[j % 2])
            if prev is not None:
                pj, pc = prev
                pc.wait()
                pltpu.sync_copy(rows_v.at[pj % 2],
                                out_hbm.at[pl.ds(base + pj * 128, 128)])
            prev = (j, cur)
        pj, pc = prev
        pc.wait()
        pltpu.sync_copy(rows_v.at[pj % 2],
                        out_hbm.at[pl.ds(base + pj * 128, 128)])

    return sc_gather, nw, b_per_w


@functools.partial(jax.jit, static_argnames=())
def kernel(z_e, emb):
    B, L, D = z_e.shape
    n = B * L
    flat = z_e.reshape(n, D)
    nblocks = n // _TB

    xsq = jnp.sum(flat ** 2, axis=1, keepdims=True)
    esq = jnp.sum(emb ** 2, axis=1, keepdims=True).T
    xsq3 = xsq.reshape(nblocks, 1, _TB)

    codes3, loss = pl.pallas_call(
        _vq_body,
        grid=(nblocks,),
        in_specs=[
            pl.BlockSpec((_TB, D), lambda i: (i, 0)),
            pl.BlockSpec((_NUM_CODES, D), lambda i: (0, 0)),
            pl.BlockSpec((1, 1, _TB), lambda i: (i, 0, 0)),
            pl.BlockSpec((1, _NUM_CODES), lambda i: (0, 0)),
        ],
        out_specs=[
            pl.BlockSpec((1, 1, _TB), lambda i: (i, 0, 0)),
            pl.BlockSpec((1, 1), lambda i: (0, 0)),
        ],
        out_shape=[
            jax.ShapeDtypeStruct((nblocks, 1, _TB), jnp.int32),
            jax.ShapeDtypeStruct((1, 1), jnp.float32),
        ],
    )(flat, emb, xsq3, esq)

    sc_gather, nw, b_per_w = _make_sc_gather(n)
    idx3 = codes3.reshape(nw, b_per_w // 128, 128)
    emb_pad = jnp.pad(emb, ((0, 0), (0, 128 - _D)))
    zq = sc_gather(emb_pad, idx3)

    codes = codes3.reshape(B, L)
    z_q = zq[:, :_D].reshape(B, L, D)
    z_q_st = z_e + lax.stop_gradient(z_q - z_e)
    loss_vq = (_COMMIT / (n * D)) * loss[0, 0]
    return (z_q_st, loss_vq, codes)
